# baseline (speedup 1.0000x reference)
import jax
import jax.numpy as jnp
from jax import lax
from jax.experimental import pallas as pl
from jax.experimental.pallas import tpu as pltpu

N = 4096
D = 64
BM = 256
NB = N // BM
QC_RES = 2560
CQ_RES = 2048
QNB = QC_RES // BM
CNB = CQ_RES // BM
QTL = NB - QNB
CTL = NB - CNB
NRF = 2
NRB = 3
TOTF = 2 * NB
TOTB = 2 * (QTL + CTL)
NTAIL = QTL + CTL


def _kernel(x0_ref, qc_ref, cq_ref, out_ref, tails_ref,
            qc16_ref, cq16_ref, fring_ref, bring_ref,
            msg_ref, xcur_ref, x16_ref, x0v_ref, outv_ref,
            fsem_ref, bsem_ref, osem_ref, iosem_ref):

    def issue_f(i):
        @pl.when(i < TOTF)
        def _():
            blk = jnp.where(i < NB, i, i - NB)
            slot = lax.rem(i, NRF)

            @pl.when(i < NB)
            def _():
                pltpu.make_async_copy(qc_ref.at[pl.ds(blk * BM, BM), :],
                                      fring_ref.at[slot],
                                      fsem_ref.at[slot]).start()

            @pl.when(i >= NB)
            def _():
                pltpu.make_async_copy(cq_ref.at[pl.ds(blk * BM, BM), :],
                                      fring_ref.at[slot],
                                      fsem_ref.at[slot]).start()

    def wait_f(slot):
        pltpu.make_async_copy(qc_ref.at[pl.ds(0, BM), :],
                              fring_ref.at[slot], fsem_ref.at[slot]).wait()

    def issue_b(q):
        if q < TOTB:
            blk = q if q < NTAIL else q - NTAIL
            slot = q % NRB
            pltpu.make_async_copy(tails_ref.at[pl.ds(blk * BM, BM), :],
                                  bring_ref.at[slot],
                                  bsem_ref.at[slot]).start()

    def wait_b(slot):
        pltpu.make_async_copy(tails_ref.at[pl.ds(0, BM), :],
                              bring_ref.at[slot], bsem_ref.at[slot]).wait()

    x0_copy = pltpu.make_async_copy(x0_ref, x0v_ref, iosem_ref.at[0])
    x0_copy.start()
    for i in range(NRF):
        issue_f(jnp.int32(i))
    x0_copy.wait()
    x16_ref[...] = x0v_ref[...].astype(jnp.bfloat16)

    def p0(j, _):
        slot = lax.rem(j, NRF)
        wait_f(slot)
        rows = pl.ds(j * BM, BM)

        @pl.when(j < QNB)
        def _():
            qc16_ref[rows, :] = fring_ref[slot].astype(jnp.bfloat16)
            msg_ref[rows, :] = jnp.dot(
                qc16_ref[rows, :], x16_ref[...],
                preferred_element_type=jnp.float32).astype(jnp.bfloat16)

        @pl.when(j >= QNB)
        def _():
            m = j - QNB
            oslot = lax.rem(m, NRB)

            @pl.when(m >= NRB)
            def _():
                pltpu.make_async_copy(bring_ref.at[oslot],
                                      tails_ref.at[pl.ds(0, BM), :],
                                      osem_ref.at[oslot]).wait()

            bring_ref[oslot] = fring_ref[slot].astype(jnp.bfloat16)
            msg_ref[rows, :] = jnp.dot(
                bring_ref[oslot], x16_ref[...],
                preferred_element_type=jnp.float32).astype(jnp.bfloat16)
            pltpu.make_async_copy(bring_ref.at[oslot],
                                  tails_ref.at[pl.ds(m * BM, BM), :],
                                  osem_ref.at[oslot]).start()

        outv_ref[rows, :] = x0v_ref[rows, :]
        issue_f(j + NRF)
        return 0

    lax.fori_loop(0, NB, p0, 0)

    def p1(j, _):
        slot = lax.rem(NB + j, NRF)
        wait_f(slot)
        rows = pl.ds(j * BM, BM)

        @pl.when(j < CNB)
        def _():
            cq16_ref[rows, :] = fring_ref[slot].astype(jnp.bfloat16)
            t = jnp.dot(cq16_ref[rows, :], msg_ref[...],
                        preferred_element_type=jnp.float32) + x0v_ref[rows, :]
            xcur_ref[rows, :] = t
            outv_ref[rows, :] += t

        @pl.when(j >= CNB)
        def _():
            m = j - CNB + QTL
            oslot = lax.rem(m, NRB)
            pltpu.make_async_copy(bring_ref.at[oslot],
                                  tails_ref.at[pl.ds(0, BM), :],
                                  osem_ref.at[oslot]).wait()
            bring_ref[oslot] = fring_ref[slot].astype(jnp.bfloat16)
            t = jnp.dot(bring_ref[oslot], msg_ref[...],
                        preferred_element_type=jnp.float32) + x0v_ref[rows, :]
            xcur_ref[rows, :] = t
            outv_ref[rows, :] += t
            pltpu.make_async_copy(bring_ref.at[oslot],
                                  tails_ref.at[pl.ds(m * BM, BM), :],
                                  osem_ref.at[oslot]).start()

        issue_f(NB + j + NRF)
        return 0

    lax.fori_loop(0, NB, p1, 0)

    for s in range(NRB):
        pltpu.make_async_copy(bring_ref.at[s],
                              tails_ref.at[pl.ds(0, BM), :],
                              osem_ref.at[s]).wait()
    for q in range(NRB):
        issue_b(q)

    def qc_phase(base):
        x16_ref[...] = xcur_ref[...].astype(jnp.bfloat16)

        def res_dot(j):
            rows = pl.ds(j * BM, BM)
            msg_ref[rows, :] = jnp.dot(
                qc16_ref[rows, :], x16_ref[...],
                preferred_element_type=jnp.float32).astype(jnp.bfloat16)

        def tail_dot(k):
            q = base + k
            slot = q % NRB
            wait_b(slot)
            rows = pl.ds((QNB + k) * BM, BM)
            msg_ref[rows, :] = jnp.dot(
                bring_ref[slot], x16_ref[...],
                preferred_element_type=jnp.float32).astype(jnp.bfloat16)
            issue_b(q + NRB)

        r = 0
        for k in range(QTL):
            n = (QNB * (k + 1) + QTL - 1) // QTL - (QNB * k + QTL - 1) // QTL
            for _ in range(n):
                if r < QNB:
                    res_dot(r)
                    r += 1
            tail_dot(k)
        while r < QNB:
            res_dot(r)
            r += 1

    def cq_phase(base, last):
        def epi(rows, t):
            if last:
                outv_ref[rows, :] = (outv_ref[rows, :] + t) * 0.25
            else:
                xcur_ref[rows, :] = t
                outv_ref[rows, :] += t

        def res_dot(j):
            rows = pl.ds(j * BM, BM)
            t = jnp.dot(cq16_ref[rows, :], msg_ref[...],
                        preferred_element_type=jnp.float32) + xcur_ref[rows, :]
            epi(rows, t)

        def tail_dot(k):
            q = base + k
            slot = q % NRB
            wait_b(slot)
            rows = pl.ds((CNB + k) * BM, BM)
            t = jnp.dot(bring_ref[slot], msg_ref[...],
                        preferred_element_type=jnp.float32) + xcur_ref[rows, :]
            epi(rows, t)
            issue_b(q + NRB)

        r = 0
        for k in range(CTL):
            n = (CNB * (k + 1) + CTL - 1) // CTL - (CNB * k + CTL - 1) // CTL
            for _ in range(n):
                if r < CNB:
                    res_dot(r)
                    r += 1
            tail_dot(k)
        while r < CNB:
            res_dot(r)
            r += 1

    qc_phase(0)
    cq_phase(QTL, False)
    qc_phase(NTAIL)
    cq_phase(NTAIL + QTL, True)

    out_copy = pltpu.make_async_copy(outv_ref, out_ref, iosem_ref.at[1])
    out_copy.start()
    out_copy.wait()


def kernel(skill_embs, HG_qc, HG_cq):
    out, _ = pl.pallas_call(
        _kernel,
        in_specs=[
            pl.BlockSpec(memory_space=pltpu.MemorySpace.HBM),
            pl.BlockSpec(memory_space=pltpu.MemorySpace.HBM),
            pl.BlockSpec(memory_space=pltpu.MemorySpace.HBM),
        ],
        out_specs=[
            pl.BlockSpec(memory_space=pltpu.MemorySpace.HBM),
            pl.BlockSpec(memory_space=pltpu.MemorySpace.HBM),
        ],
        out_shape=[
            jax.ShapeDtypeStruct((N, D), jnp.float32),
            jax.ShapeDtypeStruct((NTAIL * BM, N), jnp.bfloat16),
        ],
        compiler_params=pltpu.CompilerParams(vmem_limit_bytes=66584576),
        scratch_shapes=[
            pltpu.VMEM((QC_RES, N), jnp.bfloat16),
            pltpu.VMEM((CQ_RES, N), jnp.bfloat16),
            pltpu.VMEM((NRF, BM, N), jnp.float32),
            pltpu.VMEM((NRB, BM, N), jnp.bfloat16),
            pltpu.VMEM((N, D), jnp.bfloat16),
            pltpu.VMEM((N, D), jnp.float32),
            pltpu.VMEM((N, D), jnp.bfloat16),
            pltpu.VMEM((N, D), jnp.float32),
            pltpu.VMEM((N, D), jnp.float32),
            pltpu.SemaphoreType.DMA((NRF,)),
            pltpu.SemaphoreType.DMA((NRB,)),
            pltpu.SemaphoreType.DMA((NRB,)),
            pltpu.SemaphoreType.DMA((2,)),
        ],
    )(skill_embs, HG_qc, HG_cq)
    return out


# speedup vs baseline: 1.2200x; 1.2200x over previous
import jax
import jax.numpy as jnp
from jax import lax
from jax.experimental import pallas as pl
from jax.experimental.pallas import tpu as pltpu

N = 4096
D = 64
BM = 256
NB = N // BM
QC_RES = 2560
CQ_RES = 2560
QNB = QC_RES // BM
CNB = CQ_RES // BM
QTL = NB - QNB
CTL = NB - CNB
NR = 3

S1 = NB
S2 = S1 + NB
S3 = S2 + QTL
S4 = S3 + CTL
S5 = S4 + QTL
TOT = S5 + CTL


def _kernel(x0_ref, qc_ref, cq_ref, out_ref,
            qc16_ref, cq16_ref, ring_ref, msg_ref, xcur_ref, x16_ref,
            x0v_ref, outv_ref, sem_ref, iosem_ref):

    def issue(i):
        @pl.when(i < TOT)
        def _():
            is_qc = (i < S1) | ((i >= S2) & (i < S3)) | ((i >= S4) & (i < S5))
            blk = jnp.where(i < S1, i,
                  jnp.where(i < S2, i - S1,
                  jnp.where(i < S3, i - S2 + QNB,
                  jnp.where(i < S4, i - S3 + CNB,
                  jnp.where(i < S5, i - S4 + QNB, i - S5 + CNB)))))
            slot = lax.rem(i, NR)

            @pl.when(is_qc)
            def _():
                pltpu.make_async_copy(qc_ref.at[pl.ds(blk * BM, BM), :],
                                      ring_ref.at[slot],
                                      sem_ref.at[slot]).start()

            @pl.when(jnp.logical_not(is_qc))
            def _():
                pltpu.make_async_copy(cq_ref.at[pl.ds(blk * BM, BM), :],
                                      ring_ref.at[slot],
                                      sem_ref.at[slot]).start()

    def wait(slot):
        pltpu.make_async_copy(qc_ref.at[pl.ds(0, BM), :],
                              ring_ref.at[slot], sem_ref.at[slot]).wait()

    x0_copy = pltpu.make_async_copy(x0_ref, x0v_ref, iosem_ref.at[0])
    x0_copy.start()
    for i in range(NR):
        issue(jnp.int32(i))
    x0_copy.wait()
    x16_ref[...] = x0v_ref[...].astype(jnp.bfloat16)

    def p0(j, _):
        slot = lax.rem(j, NR)
        wait(slot)
        rows = pl.ds(j * BM, BM)

        @pl.when(j < QNB)
        def _():
            qc16_ref[rows, :] = ring_ref[slot].astype(jnp.bfloat16)
            msg_ref[rows, :] = jnp.dot(
                qc16_ref[rows, :], x16_ref[...],
                preferred_element_type=jnp.float32).astype(jnp.bfloat16)

        @pl.when(j >= QNB)
        def _():
            msg_ref[rows, :] = jnp.dot(
                ring_ref[slot].astype(jnp.bfloat16), x16_ref[...],
                preferred_element_type=jnp.float32).astype(jnp.bfloat16)

        outv_ref[rows, :] = x0v_ref[rows, :]
        issue(j + NR)
        return 0

    lax.fori_loop(0, NB, p0, 0)

    def p1(j, _):
        slot = lax.rem(S1 + j, NR)
        wait(slot)
        rows = pl.ds(j * BM, BM)

        @pl.when(j < CNB)
        def _():
            cq16_ref[rows, :] = ring_ref[slot].astype(jnp.bfloat16)
            t = jnp.dot(cq16_ref[rows, :], msg_ref[...],
                        preferred_element_type=jnp.float32) + x0v_ref[rows, :]
            xcur_ref[rows, :] = t
            outv_ref[rows, :] += t

        @pl.when(j >= CNB)
        def _():
            t = jnp.dot(ring_ref[slot].astype(jnp.bfloat16), msg_ref[...],
                        preferred_element_type=jnp.float32) + x0v_ref[rows, :]
            xcur_ref[rows, :] = t
            outv_ref[rows, :] += t

        issue(S1 + j + NR)
        return 0

    lax.fori_loop(0, NB, p1, 0)

    def qc_phase(base):
        x16_ref[...] = xcur_ref[...].astype(jnp.bfloat16)

        def res_dot(j):
            rows = pl.ds(j * BM, BM)
            msg_ref[rows, :] = jnp.dot(
                qc16_ref[rows, :], x16_ref[...],
                preferred_element_type=jnp.float32).astype(jnp.bfloat16)

        def tail_dot(k):
            i = base + k
            slot = (base + k) % NR
            wait(slot)
            b16 = ring_ref[slot].astype(jnp.bfloat16)
            rows = pl.ds((QNB + k) * BM, BM)
            msg_ref[rows, :] = jnp.dot(
                b16, x16_ref[...],
                preferred_element_type=jnp.float32).astype(jnp.bfloat16)
            issue(i + NR)

        r = 0
        for k in range(QTL):
            n = (QNB * (k + 1) + QTL - 1) // QTL - (QNB * k + QTL - 1) // QTL
            for _ in range(n):
                if r < QNB:
                    res_dot(r)
                    r += 1
            tail_dot(k)
        while r < QNB:
            res_dot(r)
            r += 1

    def cq_phase(base, last):
        def epi(rows, t):
            if last:
                outv_ref[rows, :] = (outv_ref[rows, :] + t) * 0.25
            else:
                xcur_ref[rows, :] = t
                outv_ref[rows, :] += t

        def res_dot(j):
            rows = pl.ds(j * BM, BM)
            t = jnp.dot(cq16_ref[rows, :], msg_ref[...],
                        preferred_element_type=jnp.float32) + xcur_ref[rows, :]
            epi(rows, t)

        def tail_dot(k):
            i = base + k
            slot = (base + k) % NR
            wait(slot)
            b16 = ring_ref[slot].astype(jnp.bfloat16)
            rows = pl.ds((CNB + k) * BM, BM)
            t = jnp.dot(b16, msg_ref[...],
                        preferred_element_type=jnp.float32) + xcur_ref[rows, :]
            epi(rows, t)
            issue(i + NR)

        r = 0
        for k in range(CTL):
            n = (CNB * (k + 1) + CTL - 1) // CTL - (CNB * k + CTL - 1) // CTL
            for _ in range(n):
                if r < CNB:
                    res_dot(r)
                    r += 1
            tail_dot(k)
        while r < CNB:
            res_dot(r)
            r += 1

    qc_phase(S2)
    cq_phase(S3, False)
    qc_phase(S4)
    cq_phase(S5, True)

    out_copy = pltpu.make_async_copy(outv_ref, out_ref, iosem_ref.at[1])
    out_copy.start()
    out_copy.wait()


def kernel(skill_embs, HG_qc, HG_cq):
    return pl.pallas_call(
        _kernel,
        in_specs=[
            pl.BlockSpec(memory_space=pltpu.MemorySpace.HBM),
            pl.BlockSpec(memory_space=pltpu.MemorySpace.HBM),
            pl.BlockSpec(memory_space=pltpu.MemorySpace.HBM),
        ],
        out_specs=pl.BlockSpec(memory_space=pltpu.MemorySpace.HBM),
        out_shape=jax.ShapeDtypeStruct((N, D), jnp.float32),
        compiler_params=pltpu.CompilerParams(vmem_limit_bytes=66584576),
        scratch_shapes=[
            pltpu.VMEM((QC_RES, N), jnp.bfloat16),
            pltpu.VMEM((CQ_RES, N), jnp.bfloat16),
            pltpu.VMEM((NR, BM, N), jnp.float32),
            pltpu.VMEM((N, D), jnp.bfloat16),
            pltpu.VMEM((N, D), jnp.float32),
            pltpu.VMEM((N, D), jnp.bfloat16),
            pltpu.VMEM((N, D), jnp.float32),
            pltpu.VMEM((N, D), jnp.float32),
            pltpu.SemaphoreType.DMA((NR,)),
            pltpu.SemaphoreType.DMA((2,)),
        ],
    )(skill_embs, HG_qc, HG_cq)
